# baseline (speedup 1.0000x reference)
import functools

import jax
import jax.numpy as jnp
from jax import lax
from jax.experimental import pallas as pl
from jax.experimental.pallas import tpu as pltpu
from jax.experimental.pallas import tpu_sc as plsc

_NBUF = 5
_LANES = 16


@functools.lru_cache(maxsize=None)
def _make_sc_kernel(N: int, D: int, V: int):
    info = plsc.get_sparse_core_info()
    NC, NS = info.num_cores, info.num_subcores
    NW = NC * NS
    assert N % NW == 0 and D % _LANES == 0
    rows_w = N // NW
    C = 128
    assert rows_w % C == 0
    steps = rows_w // C
    assert steps % _NBUF == 0
    R = C // _LANES
    G = D // _LANES

    mesh = plsc.VectorSubcoreMesh(core_axis_name="c", subcore_axis_name="s")

    scratch = (
        [pltpu.VMEM((C, D), jnp.float32) for _ in range(_NBUF)]
        + [pltpu.VMEM((rows_w,), jnp.int32)]
        + [pltpu.VMEM((V * D,), jnp.float32)]
        + [pltpu.SemaphoreType.DMA for _ in range(2 * _NBUF + 2)]
    )

    @functools.partial(
        pl.kernel,
        mesh=mesh,
        out_type=jax.ShapeDtypeStruct((N, D), jnp.float32),
        scratch_types=scratch,
        compiler_params=pltpu.CompilerParams(needs_layout_passes=False),
    )
    def k(x_hbm, ids_hbm, tab_hbm, out_hbm, *scr):
        bufs = scr[0:_NBUF]
        ids_v = scr[_NBUF]
        tab_v = scr[_NBUF + 1]
        s_in = scr[_NBUF + 2:2 * _NBUF + 2]
        s_out = scr[2 * _NBUF + 2:3 * _NBUF + 2]
        s_pre = scr[3 * _NBUF + 2]

        wid = lax.axis_index("s") * NC + lax.axis_index("c")
        base = wid * rows_w

        pltpu.async_copy(ids_hbm.at[pl.ds(base, rows_w)], ids_v, s_pre).wait()
        pltpu.async_copy(tab_hbm, tab_v, s_pre).wait()

        def start_in(i, b):
            pltpu.async_copy(x_hbm.at[pl.ds(base + i * C, C), :],
                             bufs[b], s_in[b])

        def wait_in(b):
            pltpu.make_async_copy(x_hbm.at[pl.ds(0, C), :], bufs[b],
                                  s_in[b]).wait()

        def start_out(i, b):
            pltpu.async_copy(bufs[b], out_hbm.at[pl.ds(base + i * C, C), :],
                             s_out[b])

        def wait_out(b):
            pltpu.make_async_copy(bufs[b], out_hbm.at[pl.ds(0, C), :],
                                  s_out[b]).wait()

        lane = lax.iota(jnp.int32, _LANES)
        dnums = lax.GatherDimensionNumbers(
            offset_dims=(), collapsed_slice_dims=(0,), start_index_map=(0,))

        def dyn_gather(vec, idx):
            return lax.gather(vec, idx[:, None], dnums, slice_sizes=(1,),
                              mode=lax.GatherScatterMode.PROMISE_IN_BOUNDS)

        tab_g = [tab_v.at[pl.ds(g * _LANES, V * D - g * _LANES)]
                 for g in range(G)]

        def compute(i, b):
            buf = bufs[b]

            @plsc.parallel_loop(0, R)
            def _(v):
                ids16 = ids_v[pl.ds(i * C + v * _LANES, _LANES)]
                idbase = ids16 * D
                for r in range(_LANES):
                    id_spl = dyn_gather(
                        idbase, jnp.full((_LANES,), r, jnp.int32))
                    idx = id_spl + lane
                    row = v * _LANES + r
                    ems = [plsc.load_gather(tab_g[g], [idx])
                           for g in range(G)]
                    for g in range(G):
                        plsc.addupdate(
                            buf.at[row, pl.ds(g * _LANES, _LANES)], ems[g])

        def body(g, carry):
            for b in range(_NBUF):
                i = g * _NBUF + b

                @pl.when(jnp.logical_and(i + 1 < steps, i + 1 >= _NBUF))
                def _():
                    wait_out((b + 1) % _NBUF)

                @pl.when(i + 1 < steps)
                def _():
                    start_in(i + 1, (b + 1) % _NBUF)

                wait_in(b)
                compute(i, b)
                start_out(i, b)
            return carry

        start_in(0, 0)
        lax.fori_loop(0, steps // _NBUF, body, 0)
        for b in range(_NBUF):
            wait_out(b)

    return k


def kernel(inputs, layer_type_ids, emb_table):
    B, L, D = inputs.shape
    V = emb_table.shape[0]
    N = B * L
    x = inputs.reshape(N, D)
    ids = layer_type_ids.reshape(N).astype(jnp.int32)
    tab = emb_table.reshape(V * D)
    out = _make_sc_kernel(N, D, V)(x, ids, tab)
    return out.reshape(B, L, D)


# speedup vs baseline: 1.0153x; 1.0153x over previous
import functools

import jax
import jax.numpy as jnp
from jax import lax
from jax.experimental import pallas as pl
from jax.experimental.pallas import tpu as pltpu
from jax.experimental.pallas import tpu_sc as plsc

_NBUF = 4
_LANES = 16


@functools.lru_cache(maxsize=None)
def _make_sc_kernel(N: int, D: int, V: int):
    info = plsc.get_sparse_core_info()
    NC, NS = info.num_cores, info.num_subcores
    NW = NC * NS
    assert N % NW == 0 and D % _LANES == 0
    rows_w = N // NW
    C = 128
    assert rows_w % C == 0
    steps = rows_w // C
    assert steps % _NBUF == 0
    R = C // _LANES
    G = D // _LANES

    mesh = plsc.VectorSubcoreMesh(core_axis_name="c", subcore_axis_name="s")

    scratch = (
        [pltpu.VMEM((C, D), jnp.float32) for _ in range(_NBUF)]
        + [pltpu.VMEM((rows_w,), jnp.int32)]
        + [pltpu.VMEM((V * D,), jnp.float32)]
        + [pltpu.SemaphoreType.DMA for _ in range(2 * _NBUF + 2)]
    )

    @functools.partial(
        pl.kernel,
        mesh=mesh,
        out_type=jax.ShapeDtypeStruct((N, D), jnp.float32),
        scratch_types=scratch,
        compiler_params=pltpu.CompilerParams(needs_layout_passes=False),
    )
    def k(x_hbm, ids_hbm, tab_hbm, out_hbm, *scr):
        bufs = scr[0:_NBUF]
        ids_v = scr[_NBUF]
        tab_v = scr[_NBUF + 1]
        s_in = scr[_NBUF + 2:2 * _NBUF + 2]
        s_out = scr[2 * _NBUF + 2:3 * _NBUF + 2]
        s_pre = scr[3 * _NBUF + 2]

        wid = lax.axis_index("s") * NC + lax.axis_index("c")
        base = wid * rows_w

        pltpu.async_copy(ids_hbm.at[pl.ds(base, rows_w)], ids_v, s_pre).wait()
        pltpu.async_copy(tab_hbm, tab_v, s_pre).wait()

        def start_in(i, b):
            pltpu.async_copy(x_hbm.at[pl.ds(base + i * C, C), :],
                             bufs[b], s_in[b])

        def wait_in(b):
            pltpu.make_async_copy(x_hbm.at[pl.ds(0, C), :], bufs[b],
                                  s_in[b]).wait()

        def start_out(i, b):
            pltpu.async_copy(bufs[b], out_hbm.at[pl.ds(base + i * C, C), :],
                             s_out[b])

        def wait_out(b):
            pltpu.make_async_copy(bufs[b], out_hbm.at[pl.ds(0, C), :],
                                  s_out[b]).wait()

        lane = lax.iota(jnp.int32, _LANES)
        dnums = lax.GatherDimensionNumbers(
            offset_dims=(), collapsed_slice_dims=(0,), start_index_map=(0,))

        def dyn_gather(vec, idx):
            return lax.gather(vec, idx[:, None], dnums, slice_sizes=(1,),
                              mode=lax.GatherScatterMode.PROMISE_IN_BOUNDS)

        tab_g = [tab_v.at[pl.ds(g * _LANES, V * D - g * _LANES)]
                 for g in range(G)]

        def compute(i, b):
            buf = bufs[b]

            @plsc.parallel_loop(0, R)
            def _(v):
                ids16 = ids_v[pl.ds(i * C + v * _LANES, _LANES)]
                idbase = ids16 * D
                for r in range(_LANES):
                    id_spl = dyn_gather(
                        idbase, jnp.full((_LANES,), r, jnp.int32))
                    idx = id_spl + lane
                    row = v * _LANES + r
                    ems = [plsc.load_gather(tab_g[g], [idx])
                           for g in range(G)]
                    for g in range(G):
                        plsc.addupdate(
                            buf.at[row, pl.ds(g * _LANES, _LANES)], ems[g])

        def body(g, carry):
            for b in range(_NBUF):
                i = g * _NBUF + b

                @pl.when(jnp.logical_and(i + 1 < steps, i + 1 >= _NBUF))
                def _():
                    wait_out((b + 1) % _NBUF)

                @pl.when(i + 1 < steps)
                def _():
                    start_in(i + 1, (b + 1) % _NBUF)

                wait_in(b)
                compute(i, b)
                start_out(i, b)
            return carry

        start_in(0, 0)
        lax.fori_loop(0, steps // _NBUF, body, 0)
        for b in range(_NBUF):
            wait_out(b)

    return k


def kernel(inputs, layer_type_ids, emb_table):
    B, L, D = inputs.shape
    V = emb_table.shape[0]
    N = B * L
    x = inputs.reshape(N, D)
    ids = layer_type_ids.reshape(N).astype(jnp.int32)
    tab = emb_table.reshape(V * D)
    out = _make_sc_kernel(N, D, V)(x, ids, tab)
    return out.reshape(B, L, D)
